# Initial kernel scaffold; baseline (speedup 1.0000x reference)
import jax
import jax.numpy as jnp
from jax.experimental import pallas as pl


def kernel(feat_rgb, feat_tir, anchors_rgb_with_conf, anchors_tir_with_conf, Wg, bg, Wa, ba):
    raise NotImplementedError("write your pallas kernel here")


# speedup vs baseline: 197.9293x; 197.9293x over previous
import functools

import jax
import jax.numpy as jnp
from jax import lax
from jax.experimental import pallas as pl
from jax.experimental.pallas import tpu as pltpu
from jax.experimental.pallas import tpu_sc as plsc

B, C, H, W = 4, 384, 64, 64
N = 100
RS = 7
HW = H * W
HWB = B * HW
D_AUG = 512
NT = 8
TW = HW // NT
NWORK = 32
NPIX_TILE = HWB // NWORK
CHUNK = 64
NCHUNK = NPIX_TILE // CHUNK
CG = C // 16


def _proj_body(fr_ref, ft_ref, wr_ref, wt_ref, or_ref, ot_ref):
    f_r = fr_ref[0]
    f_t = ft_ref[0]
    lg_r = jnp.dot(wr_ref[...], f_r, preferred_element_type=jnp.float32)
    lg_t = jnp.dot(wt_ref[...], f_t, preferred_element_type=jnp.float32)
    pad = jnp.zeros((TW, D_AUG - C - 2), jnp.float32)
    or_ref[0] = jnp.concatenate([f_r.T, lg_r.T, pad], axis=1)
    ot_ref[0] = jnp.concatenate([f_t.T, lg_t.T, pad], axis=1)


def _project(feat_rgb, feat_tir, w_r, w_t):
    fr = feat_rgb.reshape(B, C, HW)
    ft = feat_tir.reshape(B, C, HW)
    out_shape = [
        jax.ShapeDtypeStruct((B, HW, D_AUG), jnp.float32),
        jax.ShapeDtypeStruct((B, HW, D_AUG), jnp.float32),
    ]
    aug_r, aug_t = pl.pallas_call(
        _proj_body,
        grid=(B, NT),
        in_specs=[
            pl.BlockSpec((1, C, TW), lambda b, t: (b, 0, t)),
            pl.BlockSpec((1, C, TW), lambda b, t: (b, 0, t)),
            pl.BlockSpec((2, C), lambda b, t: (0, 0)),
            pl.BlockSpec((2, C), lambda b, t: (0, 0)),
        ],
        out_specs=[
            pl.BlockSpec((1, TW, D_AUG), lambda b, t: (b, t, 0)),
            pl.BlockSpec((1, TW, D_AUG), lambda b, t: (b, t, 0)),
        ],
        out_shape=out_shape,
    )(fr, ft, w_r, w_t)
    return aug_r.reshape(HWB, D_AUG), aug_t.reshape(HWB, D_AUG)


def _idx_body(ar_ref, at_ref, bg_ref, ba_ref,
              ir_ref, it_ref, mr_ref, mt_ref, sel_ref, bd_ref,
              iyr_ref, ixr_ref, iyt_ref, ixt_ref, cov_ref):
    b = pl.program_id(0)
    row = lax.broadcasted_iota(jnp.int32, (H, W), 0)
    col = lax.broadcasted_iota(jnp.int32, (H, W), 1)
    iyr_ref[...] = row
    ixr_ref[...] = col
    iyt_ref[...] = row
    ixt_ref[...] = col
    cov_ref[...] = jnp.zeros((H, W), jnp.int32)

    def body(r, _):
        axr = ar_ref[0, r, 0]
        ayr = ar_ref[0, r, 1]
        axt = at_ref[0, r, 0]
        ayt = at_ref[0, r, 1]
        y0 = jnp.clip(ayr - 4, 0, H - RS)
        x0 = jnp.clip(axr - 4, 0, W - RS)
        cover = (row >= y0) & (row < y0 + RS) & (col >= x0) & (col < x0 + RS)
        iyr_ref[...] = jnp.where(cover, ayr - 3 + (row - y0), iyr_ref[...])
        ixr_ref[...] = jnp.where(cover, axr - 3 + (col - x0), ixr_ref[...])
        iyt_ref[...] = jnp.where(cover, ayt - 3 + (row - y0), iyt_ref[...])
        ixt_ref[...] = jnp.where(cover, axt - 3 + (col - x0), ixt_ref[...])
        cov_ref[...] = jnp.where(cover, 1, cov_ref[...])
        return 0

    lax.fori_loop(0, N, body, 0)
    iyr = iyr_ref[...]
    ixr = ixr_ref[...]
    iyt = iyt_ref[...]
    ixt = ixt_ref[...]
    sel = cov_ref[...] > 0
    mr = (iyr >= -1) & (ixr >= -1)
    mt = (iyt >= -1) & (ixt >= -1)
    base = b * HW
    ir_ref[0] = base + jnp.maximum(iyr, 0) * W + jnp.maximum(ixr, 0)
    it_ref[0] = base + jnp.maximum(iyt, 0) * W + jnp.maximum(ixt, 0)
    mr_ref[0] = mr.astype(jnp.float32)
    mt_ref[0] = mt.astype(jnp.float32)
    sel_ref[0] = sel.astype(jnp.float32)
    bd_ref[0] = jnp.where(sel, ba_ref[0] - ba_ref[1], bg_ref[0] - bg_ref[1])


def _indices(anc_rgb, anc_tir, bg, ba):
    out_shape = [
        jax.ShapeDtypeStruct((B, H, W), jnp.int32),
        jax.ShapeDtypeStruct((B, H, W), jnp.int32),
        jax.ShapeDtypeStruct((B, H, W), jnp.float32),
        jax.ShapeDtypeStruct((B, H, W), jnp.float32),
        jax.ShapeDtypeStruct((B, H, W), jnp.float32),
        jax.ShapeDtypeStruct((B, H, W), jnp.float32),
    ]
    outs = pl.pallas_call(
        _idx_body,
        grid=(B,),
        in_specs=[
            pl.BlockSpec((1, N, 2), lambda b: (b, 0, 0),
                         memory_space=pltpu.SMEM),
            pl.BlockSpec((1, N, 2), lambda b: (b, 0, 0),
                         memory_space=pltpu.SMEM),
            pl.BlockSpec(memory_space=pltpu.SMEM),
            pl.BlockSpec(memory_space=pltpu.SMEM),
        ],
        out_specs=[pl.BlockSpec((1, H, W), lambda b: (b, 0, 0))] * 6,
        out_shape=out_shape,
        scratch_shapes=[pltpu.VMEM((H, W), jnp.int32)] * 5,
    )(anc_rgb, anc_tir, bg, ba)
    return tuple(o.reshape(HWB) for o in outs)


def _pack_body(mr_ref, mt_ref, sel_ref, bd_ref, o_ref):
    rows = jnp.stack(
        [mr_ref[0, 0], mt_ref[0, 0], sel_ref[0, 0], bd_ref[0, 0]], axis=0)
    meta = jnp.concatenate(
        [rows.T, jnp.zeros((TW, 12), jnp.float32)], axis=1)
    o_ref[0] = meta


def _pack_meta(m_r, m_t, sel, bd):
    ins = [x.reshape(B, 1, HW) for x in (m_r, m_t, sel, bd)]
    out = pl.pallas_call(
        _pack_body,
        grid=(B, NT),
        in_specs=[pl.BlockSpec((1, 1, TW), lambda b, t: (b, 0, t))] * 4,
        out_specs=pl.BlockSpec((1, TW, 16), lambda b, t: (b, t, 0)),
        out_shape=jax.ShapeDtypeStruct((B, HW, 16), jnp.float32),
    )(*ins)
    return out.reshape(HWB, 16)


def _sc_body(aug_r, aug_t, idx_r, idx_t, meta, out_hbm,
             idxr_v, idxt_v, rows_r, rows_t, meta_v,
             out_v, sem_r, sem_t):
    wid = lax.axis_index("s") * 2 + lax.axis_index("c")
    tbase = wid * NPIX_TILE

    def chunk_body(ci, _):
        base = tbase + ci * CHUNK
        pltpu.sync_copy(idx_r.at[pl.ds(base, CHUNK)], idxr_v)
        pltpu.sync_copy(idx_t.at[pl.ds(base, CHUNK)], idxt_v)
        pltpu.sync_copy(meta.at[pl.ds(base, CHUNK)], meta_v)
        cp_r = pltpu.async_copy(aug_r.at[idxr_v], rows_r, sem_r)
        cp_t = pltpu.async_copy(aug_t.at[idxt_v], rows_t, sem_t)
        cp_r.wait()
        cp_t.wait()

        def pix_body(p, _):
            lg_r = rows_r[p, pl.ds(C, 16)]
            lg_t = rows_t[p, pl.ds(C, 16)]
            mv = meta_v[p]
            mr = mv[0]
            mt = mv[1]
            s = mv[2]
            bd = mv[3]
            d_roi = mr * lg_r[0] + mt * lg_t[0]
            d_glb = lg_r[1] + lg_t[1]
            d = jnp.where(s > 0.5, d_roi, d_glb) + bd
            dv = lax.broadcast(d, (16,))
            alpha = 1.0 / (1.0 + jnp.exp(-dv))
            wr = lax.broadcast(mr, (16,)) * alpha
            wt = lax.broadcast(mt, (16,)) * (1.0 - alpha)
            for cg in range(CG):
                vr = rows_r[p, pl.ds(cg * 16, 16)]
                vt = rows_t[p, pl.ds(cg * 16, 16)]
                out_v[p, pl.ds(cg * 16, 16)] = wr * vr + wt * vt
            return 0

        lax.fori_loop(0, CHUNK, pix_body, 0)
        pltpu.sync_copy(out_v, out_hbm.at[pl.ds(base, CHUNK)])
        return 0

    lax.fori_loop(0, NCHUNK, chunk_body, 0)


def _sc_blend(aug_r, aug_t, idx_r, idx_t, meta):
    mesh = plsc.VectorSubcoreMesh(core_axis_name="c", subcore_axis_name="s")
    run = functools.partial(
        pl.kernel,
        mesh=mesh,
        out_type=jax.ShapeDtypeStruct((HWB, C), jnp.float32),
        scratch_types=[
            pltpu.VMEM((CHUNK,), jnp.int32),
            pltpu.VMEM((CHUNK,), jnp.int32),
            pltpu.VMEM((CHUNK, D_AUG), jnp.float32),
            pltpu.VMEM((CHUNK, D_AUG), jnp.float32),
            pltpu.VMEM((CHUNK, 16), jnp.float32),
            pltpu.VMEM((CHUNK, C), jnp.float32),
            pltpu.SemaphoreType.DMA,
            pltpu.SemaphoreType.DMA,
        ],
    )(_sc_body)
    return run(aug_r, aug_t, idx_r, idx_t, meta)


def _tr_body(x_ref, o_ref):
    o_ref[0] = x_ref[0].T


def _to_nchw(x):
    out = pl.pallas_call(
        _tr_body,
        grid=(B, NT),
        in_specs=[pl.BlockSpec((1, TW, C), lambda b, t: (b, t, 0))],
        out_specs=pl.BlockSpec((1, C, TW), lambda b, t: (b, 0, t)),
        out_shape=jax.ShapeDtypeStruct((B, C, HW), jnp.float32),
    )(x.reshape(B, HW, C))
    return out.reshape(B, C, H, W)


def kernel(feat_rgb, feat_tir, anchors_rgb_with_conf, anchors_tir_with_conf,
           Wg, bg, Wa, ba):
    anc_rgb = anchors_rgb_with_conf[..., :2].astype(jnp.int32)
    anc_tir = anchors_tir_with_conf[..., :2].astype(jnp.int32)
    w_r = jnp.stack([Wa[0, :C] - Wa[1, :C], Wg[0, :C] - Wg[1, :C]], axis=0)
    w_t = jnp.stack([Wa[0, C:] - Wa[1, C:], Wg[0, C:] - Wg[1, C:]], axis=0)

    aug_r, aug_t = _project(feat_rgb, feat_tir, w_r, w_t)
    idx_r, idx_t, m_r, m_t, sel, bd = _indices(anc_rgb, anc_tir, bg, ba)
    meta = _pack_meta(m_r, m_t, sel, bd)
    blended = _sc_blend(aug_r, aug_t, idx_r, idx_t, meta)
    return _to_nchw(blended)
